# Initial kernel scaffold; baseline (speedup 1.0000x reference)
import jax
import jax.numpy as jnp
from jax.experimental import pallas as pl


def kernel(pillars, W, gamma, beta, coords_z, coords_y, coords_x, num_points, batch_idx):
    raise NotImplementedError("write your pallas kernel here")


# speedup vs baseline: 1.6158x; 1.6158x over previous
import functools

import jax
import jax.numpy as jnp
from jax import lax
from jax.experimental import pallas as pl
from jax.experimental.pallas import tpu as pltpu
from jax.experimental.pallas import tpu_sc as plsc

P = 30000
MAX_PTS = 32
NUM_PV = 64
NX, NY, NZ = 432, 496, 1
BATCH = 2
PLANE = NZ * NY * NX
OUT_FLAT = BATCH * NUM_PV * PLANE
PILLAR_X, PILLAR_Y, PILLAR_Z = 0.16, 0.16, 4.0
X_OFF, Y_OFF, Z_OFF = 0.08, -39.6, -1.0

BLK = 512
NSTEP = (P + BLK - 1) // BLK
P_PAD = NSTEP * BLK
CHUNK = 128
NCHUNK = P_PAD // CHUNK
GROUPS = NCHUNK // 4
ZCH = PLANE // 16


def _tree(arr, op, width):
    w = arr.shape[1]
    while w > width:
        w //= 2
        arr = op(arr[:, :w], arr[:, w:2 * w])
    return arr


def _pfe_body(pt4_ref, aux_ref, wc_ref, wm_ref, wn_ref,
              xmax_ref, xmin_ref, stats_ref):
    i = pl.program_id(0)
    x4 = pt4_ref[...].reshape(4, MAX_PTS * BLK)
    aux = aux_ref[...]
    y = jnp.dot(wc_ref[...], x4, preferred_element_type=jnp.float32)
    ymax = _tree(y, jnp.maximum, BLK)
    ymin = _tree(y, jnp.minimum, BLK)
    ysum = _tree(y, jnp.add, BLK)
    ysq = _tree(y * y, jnp.add, BLK)
    sum_xyz = _tree(x4[:3], jnp.add, BLK)
    mean_xyz = sum_xyz * aux[3:4]
    const = -(jnp.dot(wm_ref[...], mean_xyz, preferred_element_type=jnp.float32)
              + jnp.dot(wn_ref[...], aux[:3], preferred_element_type=jnp.float32))
    xmax_ref[...] = ymax + const
    xmin_ref[...] = ymin + const
    col = i * BLK + lax.broadcasted_iota(jnp.int32, (NUM_PV, BLK), 1)
    valid = col < P
    s1 = jnp.sum(jnp.where(valid, ysum + 32.0 * const, 0.0), axis=1)
    s2 = jnp.sum(jnp.where(valid, ysq + 2.0 * const * ysum + 32.0 * const * const, 0.0),
                 axis=1)
    @pl.when(i == 0)
    def _():
        stats_ref[...] = jnp.zeros_like(stats_ref)
    stats_ref[...] += jnp.stack([s1, s2])


def _affine_body(xmax_ref, xmin_ref, a_ref, b_ref, gpos_ref, pv_ref):
    a, b, gpos = a_ref[...], b_ref[...], gpos_ref[...]
    sel = jnp.where(gpos > 0.5, xmax_ref[...], xmin_ref[...])
    pv_ref[...] = jnp.maximum(a * sel + b, 0.0)


def _sc_scatter_body(pv_hbm, sidx_hbm, vidx_hbm, out_hbm,
                     sidx_v, vidx_v, pvrow_v, zbuf_v, idxring_v, valring_v,
                     lsem, s0, s1, s2, s3):
    sems = (s0, s1, s2, s3)
    k = lax.axis_index("c")
    s = lax.axis_index("s")
    wid = s * 2 + k
    c1 = wid * 2

    pltpu.async_copy(sidx_hbm, sidx_v, lsem).wait()
    pltpu.async_copy(vidx_hbm, vidx_v, lsem).wait()

    zeros16 = jnp.zeros((16,), jnp.float32)
    def zinit(j, carry):
        zbuf_v[pl.ds(j * 16, 16)] = zeros16
        return carry
    lax.fori_loop(0, ZCH // 16, zinit, 0)

    handles = []
    for cc in range(2):
        for bb in range(2):
            start = (bb * NUM_PV + c1 + cc) * PLANE
            for z in range(16):
                handles.append(pltpu.async_copy(
                    zbuf_v, out_hbm.at[pl.ds(start + z * ZCH, ZCH)], lsem))
    for h in handles:
        h.wait()

    for cc in range(2):
        c = c1 + cc
        pltpu.async_copy(pv_hbm.at[c], pvrow_v, lsem).wait()
        coff = jnp.full((16,), c * PLANE, jnp.int32)

        def group(g, carry, c=c, coff=coff):
            hs = []
            for bslot in range(4):
                off = (g * 4 + bslot) * CHUNK
                for i in range(CHUNK // 16):
                    vi = vidx_v[pl.ds(off + i * 16, 16)]
                    valring_v[bslot, pl.ds(i * 16, 16)] = plsc.load_gather(
                        pvrow_v, [vi])
                    si = sidx_v[pl.ds(off + i * 16, 16)] + coff
                    idxring_v[bslot, pl.ds(i * 16, 16)] = si
                hs.append(pltpu.async_copy(
                    valring_v.at[bslot], out_hbm.at[idxring_v.at[bslot]],
                    sems[bslot]))
            for h in hs:
                h.wait()
            return carry
        lax.fori_loop(0, GROUPS, group, 0)


def kernel(pillars, W, gamma, beta, coords_z, coords_y, coords_x, num_points, batch_idx):
    f32, i32 = jnp.float32, jnp.int32
    pt4 = pillars.transpose(2, 1, 0)
    cx = coords_x.astype(f32) * PILLAR_X + X_OFF
    cy = coords_y.astype(f32) * PILLAR_Y + Y_OFF
    cz = coords_z.astype(f32) * PILLAR_Z + Z_OFF
    inv = 1.0 / jnp.maximum(num_points, 1).astype(f32)
    aux = jnp.stack([cx, cy, cz, inv])
    wc = jnp.concatenate([W[:, 0:3] + W[:, 4:7] + W[:, 7:10], W[:, 3:4]], axis=1)
    wm, wn = W[:, 4:7], W[:, 7:10]

    xmaxT, xminT, stats = pl.pallas_call(
        _pfe_body,
        grid=(NSTEP,),
        in_specs=[
            pl.BlockSpec((4, MAX_PTS, BLK), lambda i: (0, 0, i)),
            pl.BlockSpec((4, BLK), lambda i: (0, i)),
            pl.BlockSpec((NUM_PV, 4), lambda i: (0, 0)),
            pl.BlockSpec((NUM_PV, 3), lambda i: (0, 0)),
            pl.BlockSpec((NUM_PV, 3), lambda i: (0, 0)),
        ],
        out_specs=[
            pl.BlockSpec((NUM_PV, BLK), lambda i: (0, i)),
            pl.BlockSpec((NUM_PV, BLK), lambda i: (0, i)),
            pl.BlockSpec((2, NUM_PV), lambda i: (0, 0)),
        ],
        out_shape=[
            jax.ShapeDtypeStruct((NUM_PV, P_PAD), f32),
            jax.ShapeDtypeStruct((NUM_PV, P_PAD), f32),
            jax.ShapeDtypeStruct((2, NUM_PV), f32),
        ],
    )(pt4, aux, wc, wm, wn)

    n = float(P * MAX_PTS)
    mean = stats[0] / n
    var = stats[1] / n - mean * mean
    a = gamma * lax.rsqrt(var + 1e-3)
    b = beta - mean * a
    gpos = (gamma >= 0.0).astype(f32)

    pvT = pl.pallas_call(
        _affine_body,
        grid=(4,),
        in_specs=[
            pl.BlockSpec((NUM_PV, P_PAD // 4), lambda i: (0, i)),
            pl.BlockSpec((NUM_PV, P_PAD // 4), lambda i: (0, i)),
            pl.BlockSpec((NUM_PV, 1), lambda i: (0, 0)),
            pl.BlockSpec((NUM_PV, 1), lambda i: (0, 0)),
            pl.BlockSpec((NUM_PV, 1), lambda i: (0, 0)),
        ],
        out_specs=pl.BlockSpec((NUM_PV, P_PAD // 4), lambda i: (0, i)),
        out_shape=jax.ShapeDtypeStruct((NUM_PV, P_PAD), f32),
    )(xmaxT, xminT, a[:, None], b[:, None], gpos[:, None])

    flat = coords_z.astype(i32) + coords_y.astype(i32) * NX + coords_x.astype(i32)
    g = batch_idx * PLANE + flat
    winner = jnp.zeros((BATCH * PLANE,), i32).at[g].set(jnp.arange(P, dtype=i32))
    vidx = winner[g]
    sidx = batch_idx * (NUM_PV * PLANE) + flat
    pad = P_PAD - P
    sidx_p = jnp.concatenate([sidx, jnp.full((pad,), sidx[-1], i32)])
    vidx_p = jnp.concatenate([vidx, jnp.full((pad,), vidx[-1], i32)])

    sc_scatter = functools.partial(
        pl.kernel,
        mesh=plsc.VectorSubcoreMesh(core_axis_name="c", subcore_axis_name="s"),
        out_type=jax.ShapeDtypeStruct((OUT_FLAT,), f32),
        compiler_params=pltpu.CompilerParams(needs_layout_passes=False),
        scratch_types=[
            pltpu.VMEM((P_PAD,), i32),
            pltpu.VMEM((P_PAD,), i32),
            pltpu.VMEM((P_PAD,), f32),
            pltpu.VMEM((ZCH,), f32),
            pltpu.VMEM((4, CHUNK), i32),
            pltpu.VMEM((4, CHUNK), f32),
            pltpu.SemaphoreType.DMA,
            pltpu.SemaphoreType.DMA,
            pltpu.SemaphoreType.DMA,
            pltpu.SemaphoreType.DMA,
            pltpu.SemaphoreType.DMA,
        ],
    )(_sc_scatter_body)
    bev1d = sc_scatter(pvT, sidx_p, vidx_p)
    return bev1d.reshape(BATCH, NUM_PV * NZ, NY, NX)


# speedup vs baseline: 2.0621x; 1.2762x over previous
import functools

import jax
import jax.numpy as jnp
from jax import lax
from jax.experimental import pallas as pl
from jax.experimental.pallas import tpu as pltpu
from jax.experimental.pallas import tpu_sc as plsc

P = 30000
MAX_PTS = 32
NUM_PV = 64
NX, NY, NZ = 432, 496, 1
BATCH = 2
PLANE = NZ * NY * NX
OUT_FLAT = BATCH * NUM_PV * PLANE
PILLAR_X, PILLAR_Y, PILLAR_Z = 0.16, 0.16, 4.0
X_OFF, Y_OFF, Z_OFF = 0.08, -39.6, -1.0

BLK = 512
NSTEP = (P + BLK - 1) // BLK
P_PAD = NSTEP * BLK
CHUNK = 128
NTILE = 32
CPT = 8
P_PAD2 = NTILE * CPT * CHUNK
TF = 256
TSTEP = PLANE // TF


def _tree(arr, op, width):
    w = arr.shape[1]
    while w > width:
        w //= 2
        arr = op(arr[:, :w], arr[:, w:2 * w])
    return arr


def _pfe_body(pt4_ref, aux_ref, wc_ref, wm_ref, wn_ref,
              xmax_ref, xmin_ref, stats_ref):
    i = pl.program_id(0)
    x4 = pt4_ref[...].reshape(4, MAX_PTS * BLK)
    aux = aux_ref[...]
    y = jnp.dot(wc_ref[...], x4, preferred_element_type=jnp.float32)
    ymax = _tree(y, jnp.maximum, BLK)
    ymin = _tree(y, jnp.minimum, BLK)
    ysum = _tree(y, jnp.add, BLK)
    ysq = _tree(y * y, jnp.add, BLK)
    sum_xyz = _tree(x4[:3], jnp.add, BLK)
    mean_xyz = sum_xyz * aux[3:4]
    const = -(jnp.dot(wm_ref[...], mean_xyz, preferred_element_type=jnp.float32)
              + jnp.dot(wn_ref[...], aux[:3], preferred_element_type=jnp.float32))
    xmax_ref[...] = ymax + const
    xmin_ref[...] = ymin + const
    col = i * BLK + lax.broadcasted_iota(jnp.int32, (NUM_PV, BLK), 1)
    valid = col < P
    s1 = jnp.sum(jnp.where(valid, ysum + 32.0 * const, 0.0), axis=1)
    s2 = jnp.sum(jnp.where(valid, ysq + 2.0 * const * ysum + 32.0 * const * const, 0.0),
                 axis=1)
    @pl.when(i == 0)
    def _():
        stats_ref[...] = jnp.zeros_like(stats_ref)
    stats_ref[...] += jnp.stack([s1, s2])


def _affine_body(xmax_ref, xmin_ref, a_ref, b_ref, gpos_ref, pv_ref):
    a, b, gpos = a_ref[...], b_ref[...], gpos_ref[...]
    sel = jnp.where(gpos > 0.5, xmax_ref[...], xmin_ref[...])
    pv_ref[...] = jnp.maximum(a * sel + b, 0.0).T


def _sc_scatter_body(pv_hbm, vidx_hbm, gidx_hbm, out_hbm,
                     vidx_v, gidx_v, row0_v, row1_v, lsem, g0, g1, s0, s1):
    k = lax.axis_index("c")
    s = lax.axis_index("s")
    wid = s * 2 + k
    rows = (row0_v, row1_v)
    gsem = (g0, g1)
    ssem = (s0, s1)
    pltpu.async_copy(vidx_hbm.at[wid], vidx_v, lsem).wait()
    pltpu.async_copy(gidx_hbm.at[wid], gidx_v, lsem).wait()
    gh = [None, None]
    sh = [None, None]
    for j in range(CPT):
        b = j % 2
        if sh[b] is not None:
            sh[b].wait()
        gh[b] = pltpu.async_copy(pv_hbm.at[vidx_v.at[j]], rows[b], gsem[b])
        gh[b].wait()
        sh[b] = pltpu.async_copy(rows[b], out_hbm.at[gidx_v.at[j]], ssem[b])
    for b in range(2):
        if sh[b] is not None:
            sh[b].wait()


def _transpose_body(x_ref, occ_ref, out_ref):
    x = x_ref[0]
    occ = occ_ref[0, 0]
    out_ref[0] = jnp.where(occ != 0, x.T, 0.0)


def kernel(pillars, W, gamma, beta, coords_z, coords_y, coords_x, num_points, batch_idx):
    f32, i32 = jnp.float32, jnp.int32
    pt4 = pillars.transpose(2, 1, 0)
    cx = coords_x.astype(f32) * PILLAR_X + X_OFF
    cy = coords_y.astype(f32) * PILLAR_Y + Y_OFF
    cz = coords_z.astype(f32) * PILLAR_Z + Z_OFF
    inv = 1.0 / jnp.maximum(num_points, 1).astype(f32)
    aux = jnp.stack([cx, cy, cz, inv])
    wc = jnp.concatenate([W[:, 0:3] + W[:, 4:7] + W[:, 7:10], W[:, 3:4]], axis=1)
    wm, wn = W[:, 4:7], W[:, 7:10]

    xmaxT, xminT, stats = pl.pallas_call(
        _pfe_body,
        grid=(NSTEP,),
        in_specs=[
            pl.BlockSpec((4, MAX_PTS, BLK), lambda i: (0, 0, i)),
            pl.BlockSpec((4, BLK), lambda i: (0, i)),
            pl.BlockSpec((NUM_PV, 4), lambda i: (0, 0)),
            pl.BlockSpec((NUM_PV, 3), lambda i: (0, 0)),
            pl.BlockSpec((NUM_PV, 3), lambda i: (0, 0)),
        ],
        out_specs=[
            pl.BlockSpec((NUM_PV, BLK), lambda i: (0, i)),
            pl.BlockSpec((NUM_PV, BLK), lambda i: (0, i)),
            pl.BlockSpec((2, NUM_PV), lambda i: (0, 0)),
        ],
        out_shape=[
            jax.ShapeDtypeStruct((NUM_PV, P_PAD), f32),
            jax.ShapeDtypeStruct((NUM_PV, P_PAD), f32),
            jax.ShapeDtypeStruct((2, NUM_PV), f32),
        ],
    )(pt4, aux, wc, wm, wn)

    n = float(P * MAX_PTS)
    mean = stats[0] / n
    var = stats[1] / n - mean * mean
    a = gamma * lax.rsqrt(var + 1e-3)
    b = beta - mean * a
    gpos = (gamma >= 0.0).astype(f32)

    pv = pl.pallas_call(
        _affine_body,
        grid=(4,),
        in_specs=[
            pl.BlockSpec((NUM_PV, P_PAD // 4), lambda i: (0, i)),
            pl.BlockSpec((NUM_PV, P_PAD // 4), lambda i: (0, i)),
            pl.BlockSpec((NUM_PV, 1), lambda i: (0, 0)),
            pl.BlockSpec((NUM_PV, 1), lambda i: (0, 0)),
            pl.BlockSpec((NUM_PV, 1), lambda i: (0, 0)),
        ],
        out_specs=pl.BlockSpec((P_PAD // 4, NUM_PV), lambda i: (i, 0)),
        out_shape=jax.ShapeDtypeStruct((P_PAD, NUM_PV), f32),
    )(xmaxT, xminT, a[:, None], b[:, None], gpos[:, None])

    flat = coords_z.astype(i32) + coords_y.astype(i32) * NX + coords_x.astype(i32)
    g = batch_idx * PLANE + flat
    winner = jnp.zeros((BATCH * PLANE,), i32).at[g].set(jnp.arange(P, dtype=i32))
    occ = jnp.zeros((BATCH * PLANE,), i32).at[g].set(1)
    vidx = winner[g]
    pad = P_PAD2 - P
    gidx_p = jnp.concatenate([g, jnp.full((pad,), g[-1], i32)]).reshape(
        NTILE, CPT, CHUNK)
    vidx_p = jnp.concatenate([vidx, jnp.full((pad,), vidx[-1], i32)]).reshape(
        NTILE, CPT, CHUNK)

    sc_scatter = functools.partial(
        pl.kernel,
        mesh=plsc.VectorSubcoreMesh(core_axis_name="c", subcore_axis_name="s"),
        out_type=jax.ShapeDtypeStruct((BATCH * PLANE, NUM_PV), f32),
        compiler_params=pltpu.CompilerParams(needs_layout_passes=False,
                                             use_tc_tiling_on_sc=False),
        scratch_types=[
            pltpu.VMEM((CPT, CHUNK), i32),
            pltpu.VMEM((CPT, CHUNK), i32),
            pltpu.VMEM((CHUNK, NUM_PV), f32),
            pltpu.VMEM((CHUNK, NUM_PV), f32),
            pltpu.SemaphoreType.DMA,
            pltpu.SemaphoreType.DMA,
            pltpu.SemaphoreType.DMA,
            pltpu.SemaphoreType.DMA,
            pltpu.SemaphoreType.DMA,
        ],
    )(_sc_scatter_body)
    scratch = sc_scatter(pv, vidx_p, gidx_p)

    bev = pl.pallas_call(
        _transpose_body,
        grid=(BATCH, TSTEP),
        in_specs=[
            pl.BlockSpec((1, TF, NUM_PV), lambda b, i: (b, i, 0)),
            pl.BlockSpec((1, 1, 1, TF), lambda b, i: (b, i, 0, 0)),
        ],
        out_specs=pl.BlockSpec((1, NUM_PV, TF), lambda b, i: (b, 0, i)),
        out_shape=jax.ShapeDtypeStruct((BATCH, NUM_PV, PLANE), f32),
    )(scratch.reshape(BATCH, PLANE, NUM_PV),
      occ.reshape(BATCH, TSTEP, 1, TF))
    return bev.reshape(BATCH, NUM_PV * NZ, NY, NX)


# speedup vs baseline: 2.1021x; 1.0194x over previous
import functools

import jax
import jax.numpy as jnp
from jax import lax
from jax.experimental import pallas as pl
from jax.experimental.pallas import tpu as pltpu
from jax.experimental.pallas import tpu_sc as plsc

P = 30000
MAX_PTS = 32
NUM_PV = 64
NX, NY, NZ = 432, 496, 1
BATCH = 2
PLANE = NZ * NY * NX
OUT_FLAT = BATCH * NUM_PV * PLANE
PILLAR_X, PILLAR_Y, PILLAR_Z = 0.16, 0.16, 4.0
X_OFF, Y_OFF, Z_OFF = 0.08, -39.6, -1.0

BLK = 512
NSTEP = (P + BLK - 1) // BLK
P_PAD = NSTEP * BLK
CHUNK = 128
NTILE = 32
CPT = 8
P_PAD2 = NTILE * CPT * CHUNK
TF = 256
TSTEP = PLANE // TF


def _tree(arr, op, width):
    w = arr.shape[1]
    while w > width:
        w //= 2
        arr = op(arr[:, :w], arr[:, w:2 * w])
    return arr


def _rtree(arr, op, stop):
    r = arr.shape[0]
    while r > stop:
        r //= 2
        arr = op(arr[:r], arr[r:2 * r])
    return arr


def _pfe_body(x_ref, aux_ref, wb_ref, wm_ref, wn_ref,
              xmax_ref, xmin_ref, stats_ref):
    i = pl.program_id(0)
    aux = aux_ref[...]
    y2 = lax.dot_general(wb_ref[...], x_ref[...], (((1,), (1,)), ((), ())),
                         preferred_element_type=jnp.float32)
    ych = y2[:2048]
    ymax = _rtree(ych, jnp.maximum, NUM_PV)
    ymin = _rtree(ych, jnp.minimum, NUM_PV)
    ysum = _rtree(ych, jnp.add, NUM_PV)
    ysq = _rtree(ych * ych, jnp.add, NUM_PV)
    sum_xyz = _rtree(y2[2048:], jnp.add, 4)[:3]
    mean_xyz = sum_xyz * aux[3:4]
    const = -(jnp.dot(wm_ref[...], mean_xyz, preferred_element_type=jnp.float32)
              + jnp.dot(wn_ref[...], aux[:3], preferred_element_type=jnp.float32))
    xmax_ref[...] = ymax + const
    xmin_ref[...] = ymin + const
    col = i * BLK + lax.broadcasted_iota(jnp.int32, (NUM_PV, BLK), 1)
    valid = col < P
    s1 = jnp.sum(jnp.where(valid, ysum + 32.0 * const, 0.0), axis=1)
    s2 = jnp.sum(jnp.where(valid, ysq + 2.0 * const * ysum + 32.0 * const * const, 0.0),
                 axis=1)
    @pl.when(i == 0)
    def _():
        stats_ref[...] = jnp.zeros_like(stats_ref)
    stats_ref[...] += jnp.stack([s1, s2])


def _affine_body(xmax_ref, xmin_ref, a_ref, b_ref, gpos_ref, pv_ref):
    a, b, gpos = a_ref[...], b_ref[...], gpos_ref[...]
    sel = jnp.where(gpos > 0.5, xmax_ref[...], xmin_ref[...])
    pv_ref[...] = jnp.maximum(a * sel + b, 0.0).T


def _sc_scatter_body(pv_hbm, vidx_hbm, gidx_hbm, out_hbm,
                     vidx_v, gidx_v, row0_v, row1_v, lsem, g0, g1, s0, s1):
    k = lax.axis_index("c")
    s = lax.axis_index("s")
    wid = s * 2 + k
    rows = (row0_v, row1_v)
    gsem = (g0, g1)
    ssem = (s0, s1)
    pltpu.async_copy(vidx_hbm.at[wid], vidx_v, lsem).wait()
    pltpu.async_copy(gidx_hbm.at[wid], gidx_v, lsem).wait()
    gh = [None, None]
    sh = [None, None]
    for j in range(CPT):
        b = j % 2
        if sh[b] is not None:
            sh[b].wait()
        gh[b] = pltpu.async_copy(pv_hbm.at[vidx_v.at[j]], rows[b], gsem[b])
        gh[b].wait()
        sh[b] = pltpu.async_copy(rows[b], out_hbm.at[gidx_v.at[j]], ssem[b])
    for b in range(2):
        if sh[b] is not None:
            sh[b].wait()


def _transpose_body(x_ref, occ_ref, out_ref):
    x = x_ref[0]
    occ = occ_ref[0, 0]
    out_ref[0] = jnp.where(occ != 0, x.T, 0.0)


def kernel(pillars, W, gamma, beta, coords_z, coords_y, coords_x, num_points, batch_idx):
    f32, i32 = jnp.float32, jnp.int32
    x128 = pillars.reshape(P, MAX_PTS * 4)
    cx = coords_x.astype(f32) * PILLAR_X + X_OFF
    cy = coords_y.astype(f32) * PILLAR_Y + Y_OFF
    cz = coords_z.astype(f32) * PILLAR_Z + Z_OFF
    inv = 1.0 / jnp.maximum(num_points, 1).astype(f32)
    aux = jnp.stack([cx, cy, cz, inv])
    wc = jnp.concatenate([W[:, 0:3] + W[:, 4:7] + W[:, 7:10], W[:, 3:4]], axis=1)
    wm, wn = W[:, 4:7], W[:, 7:10]
    eye32 = jnp.eye(MAX_PTS, dtype=f32)
    wb = jnp.concatenate([jnp.kron(eye32, wc),
                          jnp.kron(eye32, jnp.eye(4, dtype=f32))], axis=0)

    xmaxT, xminT, stats = pl.pallas_call(
        _pfe_body,
        grid=(NSTEP,),
        in_specs=[
            pl.BlockSpec((BLK, MAX_PTS * 4), lambda i: (i, 0)),
            pl.BlockSpec((4, BLK), lambda i: (0, i)),
            pl.BlockSpec((2176, MAX_PTS * 4), lambda i: (0, 0)),
            pl.BlockSpec((NUM_PV, 3), lambda i: (0, 0)),
            pl.BlockSpec((NUM_PV, 3), lambda i: (0, 0)),
        ],
        out_specs=[
            pl.BlockSpec((NUM_PV, BLK), lambda i: (0, i)),
            pl.BlockSpec((NUM_PV, BLK), lambda i: (0, i)),
            pl.BlockSpec((2, NUM_PV), lambda i: (0, 0)),
        ],
        out_shape=[
            jax.ShapeDtypeStruct((NUM_PV, P_PAD), f32),
            jax.ShapeDtypeStruct((NUM_PV, P_PAD), f32),
            jax.ShapeDtypeStruct((2, NUM_PV), f32),
        ],
    )(x128, aux, wb, wm, wn)

    n = float(P * MAX_PTS)
    mean = stats[0] / n
    var = stats[1] / n - mean * mean
    a = gamma * lax.rsqrt(var + 1e-3)
    b = beta - mean * a
    gpos = (gamma >= 0.0).astype(f32)

    pv = pl.pallas_call(
        _affine_body,
        grid=(4,),
        in_specs=[
            pl.BlockSpec((NUM_PV, P_PAD // 4), lambda i: (0, i)),
            pl.BlockSpec((NUM_PV, P_PAD // 4), lambda i: (0, i)),
            pl.BlockSpec((NUM_PV, 1), lambda i: (0, 0)),
            pl.BlockSpec((NUM_PV, 1), lambda i: (0, 0)),
            pl.BlockSpec((NUM_PV, 1), lambda i: (0, 0)),
        ],
        out_specs=pl.BlockSpec((P_PAD // 4, NUM_PV), lambda i: (i, 0)),
        out_shape=jax.ShapeDtypeStruct((P_PAD, NUM_PV), f32),
    )(xmaxT, xminT, a[:, None], b[:, None], gpos[:, None])

    flat = coords_z.astype(i32) + coords_y.astype(i32) * NX + coords_x.astype(i32)
    g = batch_idx * PLANE + flat
    winner = jnp.full((BATCH * PLANE,), -1, i32).at[g].set(jnp.arange(P, dtype=i32))
    occ = (winner >= 0).astype(i32)
    vidx = winner[g]
    pad = P_PAD2 - P
    gidx_p = jnp.concatenate([g, jnp.full((pad,), g[-1], i32)]).reshape(
        NTILE, CPT, CHUNK)
    vidx_p = jnp.concatenate([vidx, jnp.full((pad,), vidx[-1], i32)]).reshape(
        NTILE, CPT, CHUNK)

    sc_scatter = functools.partial(
        pl.kernel,
        mesh=plsc.VectorSubcoreMesh(core_axis_name="c", subcore_axis_name="s"),
        out_type=jax.ShapeDtypeStruct((BATCH * PLANE, NUM_PV), f32),
        compiler_params=pltpu.CompilerParams(needs_layout_passes=False,
                                             use_tc_tiling_on_sc=False),
        scratch_types=[
            pltpu.VMEM((CPT, CHUNK), i32),
            pltpu.VMEM((CPT, CHUNK), i32),
            pltpu.VMEM((CHUNK, NUM_PV), f32),
            pltpu.VMEM((CHUNK, NUM_PV), f32),
            pltpu.SemaphoreType.DMA,
            pltpu.SemaphoreType.DMA,
            pltpu.SemaphoreType.DMA,
            pltpu.SemaphoreType.DMA,
            pltpu.SemaphoreType.DMA,
        ],
    )(_sc_scatter_body)
    scratch = sc_scatter(pv, vidx_p, gidx_p)

    bev = pl.pallas_call(
        _transpose_body,
        grid=(BATCH, TSTEP),
        in_specs=[
            pl.BlockSpec((1, TF, NUM_PV), lambda b, i: (b, i, 0)),
            pl.BlockSpec((1, 1, 1, TF), lambda b, i: (b, i, 0, 0)),
        ],
        out_specs=pl.BlockSpec((1, NUM_PV, TF), lambda b, i: (b, 0, i)),
        out_shape=jax.ShapeDtypeStruct((BATCH, NUM_PV, PLANE), f32),
    )(scratch.reshape(BATCH, PLANE, NUM_PV),
      occ.reshape(BATCH, TSTEP, 1, TF))
    return bev.reshape(BATCH, NUM_PV * NZ, NY, NX)


# speedup vs baseline: 2.2143x; 1.0534x over previous
import functools

import jax
import jax.numpy as jnp
from jax import lax
from jax.experimental import pallas as pl
from jax.experimental.pallas import tpu as pltpu
from jax.experimental.pallas import tpu_sc as plsc

P = 30000
MAX_PTS = 32
NUM_PV = 64
NX, NY, NZ = 432, 496, 1
BATCH = 2
PLANE = NZ * NY * NX
OUT_FLAT = BATCH * NUM_PV * PLANE
PILLAR_X, PILLAR_Y, PILLAR_Z = 0.16, 0.16, 4.0
X_OFF, Y_OFF, Z_OFF = 0.08, -39.6, -1.0

BLK = 512
NSTEP = (P + BLK - 1) // BLK
P_PAD = NSTEP * BLK
CHUNK = 128
NTILE = 32
CPT = 8
P_PAD2 = NTILE * CPT * CHUNK
TF = 256
TSTEP = PLANE // TF


def _tree(arr, op, width):
    w = arr.shape[1]
    while w > width:
        w //= 2
        arr = op(arr[:, :w], arr[:, w:2 * w])
    return arr


def _rtree(arr, op, stop):
    r = arr.shape[0]
    while r > stop:
        r //= 2
        arr = op(arr[:r], arr[r:2 * r])
    return arr


def _pfe_body(x_ref, aux_ref, wb_ref, wm_ref, wn_ref,
              xmax_ref, xmin_ref, stats_ref):
    i = pl.program_id(0)
    aux = aux_ref[...]
    y2 = lax.dot_general(wb_ref[...], x_ref[...], (((1,), (1,)), ((), ())),
                         preferred_element_type=jnp.float32)
    ych = y2[:2048]
    ymax = _rtree(ych, jnp.maximum, NUM_PV)
    ymin = _rtree(ych, jnp.minimum, NUM_PV)
    ysum = _rtree(ych, jnp.add, NUM_PV)
    ysq = _rtree(ych * ych, jnp.add, NUM_PV)
    sum_xyz = _rtree(y2[2048:], jnp.add, 4)[:3]
    mean_xyz = sum_xyz * aux[3:4]
    const = -(jnp.dot(wm_ref[...], mean_xyz, preferred_element_type=jnp.float32)
              + jnp.dot(wn_ref[...], aux[:3], preferred_element_type=jnp.float32))
    xmax_ref[...] = ymax + const
    xmin_ref[...] = ymin + const
    col = i * BLK + lax.broadcasted_iota(jnp.int32, (NUM_PV, BLK), 1)
    valid = col < P
    s1 = jnp.sum(jnp.where(valid, ysum + 32.0 * const, 0.0), axis=1)
    s2 = jnp.sum(jnp.where(valid, ysq + 2.0 * const * ysum + 32.0 * const * const, 0.0),
                 axis=1)
    @pl.when(i == 0)
    def _():
        stats_ref[...] = jnp.zeros_like(stats_ref)
    stats_ref[...] += jnp.stack([s1, s2])


def _affine_body(xmax_ref, xmin_ref, a_ref, b_ref, gpos_ref, pv_ref):
    a, b, gpos = a_ref[...], b_ref[...], gpos_ref[...]
    sel = jnp.where(gpos > 0.5, xmax_ref[...], xmin_ref[...])
    pvt = jnp.maximum(a * sel + b, 0.0).T
    pv_ref[...] = jnp.pad(pvt, ((0, 0), (0, NUM_PV)))


def _sc_scatter_body(pv_hbm, vidx_hbm, gidx_hbm, out_hbm,
                     vidx_v, gidx_v, row0_v, row1_v, lsem, g0, g1, s0, s1):
    k = lax.axis_index("c")
    s = lax.axis_index("s")
    wid = s * 2 + k
    rows = (row0_v, row1_v)
    gsem = (g0, g1)
    ssem = (s0, s1)
    pltpu.async_copy(vidx_hbm.at[wid], vidx_v, lsem).wait()
    pltpu.async_copy(gidx_hbm.at[wid], gidx_v, lsem).wait()
    gh = [None, None]
    sh = [None, None]
    for j in range(CPT):
        b = j % 2
        if sh[b] is not None:
            sh[b].wait()
        gh[b] = pltpu.async_copy(pv_hbm.at[vidx_v.at[j]], rows[b], gsem[b])
        gh[b].wait()
        sh[b] = pltpu.async_copy(rows[b], out_hbm.at[gidx_v.at[j]], ssem[b])
    for b in range(2):
        if sh[b] is not None:
            sh[b].wait()


def _transpose_body(x_ref, occ_ref, out_ref):
    x = x_ref[0, :, :NUM_PV]
    occ = occ_ref[0, 0]
    out_ref[0] = jnp.where(occ != 0, x.T, 0.0)


def kernel(pillars, W, gamma, beta, coords_z, coords_y, coords_x, num_points, batch_idx):
    f32, i32 = jnp.float32, jnp.int32
    x128 = pillars.reshape(P, MAX_PTS * 4)
    cx = coords_x.astype(f32) * PILLAR_X + X_OFF
    cy = coords_y.astype(f32) * PILLAR_Y + Y_OFF
    cz = coords_z.astype(f32) * PILLAR_Z + Z_OFF
    inv = 1.0 / jnp.maximum(num_points, 1).astype(f32)
    aux = jnp.stack([cx, cy, cz, inv])
    wc = jnp.concatenate([W[:, 0:3] + W[:, 4:7] + W[:, 7:10], W[:, 3:4]], axis=1)
    wm, wn = W[:, 4:7], W[:, 7:10]
    eye32 = jnp.eye(MAX_PTS, dtype=f32)
    wb = jnp.concatenate([jnp.kron(eye32, wc),
                          jnp.kron(eye32, jnp.eye(4, dtype=f32))], axis=0)

    xmaxT, xminT, stats = pl.pallas_call(
        _pfe_body,
        grid=(NSTEP,),
        in_specs=[
            pl.BlockSpec((BLK, MAX_PTS * 4), lambda i: (i, 0)),
            pl.BlockSpec((4, BLK), lambda i: (0, i)),
            pl.BlockSpec((2176, MAX_PTS * 4), lambda i: (0, 0)),
            pl.BlockSpec((NUM_PV, 3), lambda i: (0, 0)),
            pl.BlockSpec((NUM_PV, 3), lambda i: (0, 0)),
        ],
        out_specs=[
            pl.BlockSpec((NUM_PV, BLK), lambda i: (0, i)),
            pl.BlockSpec((NUM_PV, BLK), lambda i: (0, i)),
            pl.BlockSpec((2, NUM_PV), lambda i: (0, 0)),
        ],
        out_shape=[
            jax.ShapeDtypeStruct((NUM_PV, P_PAD), f32),
            jax.ShapeDtypeStruct((NUM_PV, P_PAD), f32),
            jax.ShapeDtypeStruct((2, NUM_PV), f32),
        ],
    )(x128, aux, wb, wm, wn)

    n = float(P * MAX_PTS)
    mean = stats[0] / n
    var = stats[1] / n - mean * mean
    a = gamma * lax.rsqrt(var + 1e-3)
    b = beta - mean * a
    gpos = (gamma >= 0.0).astype(f32)

    pv = pl.pallas_call(
        _affine_body,
        grid=(4,),
        in_specs=[
            pl.BlockSpec((NUM_PV, P_PAD // 4), lambda i: (0, i)),
            pl.BlockSpec((NUM_PV, P_PAD // 4), lambda i: (0, i)),
            pl.BlockSpec((NUM_PV, 1), lambda i: (0, 0)),
            pl.BlockSpec((NUM_PV, 1), lambda i: (0, 0)),
            pl.BlockSpec((NUM_PV, 1), lambda i: (0, 0)),
        ],
        out_specs=pl.BlockSpec((P_PAD // 4, 2 * NUM_PV), lambda i: (i, 0)),
        out_shape=jax.ShapeDtypeStruct((P_PAD, 2 * NUM_PV), f32),
    )(xmaxT, xminT, a[:, None], b[:, None], gpos[:, None])

    flat = coords_z.astype(i32) + coords_y.astype(i32) * NX + coords_x.astype(i32)
    g = batch_idx * PLANE + flat
    winner = jnp.full((BATCH * PLANE,), -1, i32).at[g].set(jnp.arange(P, dtype=i32))
    occ = (winner >= 0).astype(i32)
    vidx = winner[g]
    pad = P_PAD2 - P
    gidx_p = jnp.concatenate([g, jnp.full((pad,), g[-1], i32)]).reshape(
        NTILE, CPT, CHUNK)
    vidx_p = jnp.concatenate([vidx, jnp.full((pad,), vidx[-1], i32)]).reshape(
        NTILE, CPT, CHUNK)

    sc_scatter = functools.partial(
        pl.kernel,
        mesh=plsc.VectorSubcoreMesh(core_axis_name="c", subcore_axis_name="s"),
        out_type=jax.ShapeDtypeStruct((BATCH * PLANE, 2 * NUM_PV), f32),
        compiler_params=pltpu.CompilerParams(needs_layout_passes=False),
        scratch_types=[
            pltpu.VMEM((CPT, CHUNK), i32),
            pltpu.VMEM((CPT, CHUNK), i32),
            pltpu.VMEM((CHUNK, 2 * NUM_PV), f32),
            pltpu.VMEM((CHUNK, 2 * NUM_PV), f32),
            pltpu.SemaphoreType.DMA,
            pltpu.SemaphoreType.DMA,
            pltpu.SemaphoreType.DMA,
            pltpu.SemaphoreType.DMA,
            pltpu.SemaphoreType.DMA,
        ],
    )(_sc_scatter_body)
    scratch = sc_scatter(pv, vidx_p, gidx_p)

    bev = pl.pallas_call(
        _transpose_body,
        grid=(BATCH, TSTEP),
        in_specs=[
            pl.BlockSpec((1, TF, 2 * NUM_PV), lambda b, i: (b, i, 0)),
            pl.BlockSpec((1, 1, 1, TF), lambda b, i: (b, i, 0, 0)),
        ],
        out_specs=pl.BlockSpec((1, NUM_PV, TF), lambda b, i: (b, 0, i)),
        out_shape=jax.ShapeDtypeStruct((BATCH, NUM_PV, PLANE), f32),
    )(scratch.reshape(BATCH, PLANE, 2 * NUM_PV),
      occ.reshape(BATCH, TSTEP, 1, TF))
    return bev.reshape(BATCH, NUM_PV * NZ, NY, NX)
